# Initial kernel scaffold; baseline (speedup 1.0000x reference)
import functools

import jax
import jax.numpy as jnp
from jax import lax
from jax.experimental import pallas as pl
from jax.experimental.pallas import tpu as pltpu
from jax.experimental.pallas import tpu_sc as plsc

DIM = 32
NC = 2
NS = 16
NW = NC * NS

GATHER = 128
GPG = 10
GROUP = GATHER * GPG


def _emb_body(table_hbm, idx_hbm, out_hbm, idx_v, rows_v, sem):
    n_groups = idx_hbm.shape[1]
    wid = lax.axis_index("s") * NC + lax.axis_index("c")

    def group(g, carry):
        pltpu.sync_copy(idx_hbm.at[wid, g], idx_v)
        handles = [
            pltpu.async_copy(
                table_hbm.at[idx_v.at[j]],
                rows_v.at[pl.ds(j * GATHER, GATHER)],
                sem,
            )
            for j in range(GPG)
        ]
        for h in handles:
            h.wait()
        pltpu.sync_copy(rows_v, out_hbm.at[wid, g])
        return carry

    lax.fori_loop(0, n_groups, group, 0)


def _make_call(n_groups):
    return pl.kernel(
        _emb_body,
        out_type=jax.ShapeDtypeStruct((NW, n_groups, GROUP, DIM), jnp.float32),
        mesh=plsc.VectorSubcoreMesh(core_axis_name="c", subcore_axis_name="s"),
        scratch_types=[
            pltpu.VMEM((GPG, GATHER), jnp.int32),
            pltpu.VMEM((GROUP, DIM), jnp.float32),
            pltpu.SemaphoreType.DMA,
        ],
    )


def kernel(x, table):
    b, s = x.shape
    total = b * s
    assert total % (NW * GROUP) == 0
    n_groups = total // (NW * GROUP)
    idx = x.reshape(NW, n_groups, GPG, GATHER).astype(jnp.int32)
    out = _make_call(n_groups)(table, idx)
    return out.reshape(b, s, DIM)


# speedup vs baseline: 1.2681x; 1.2681x over previous
import functools

import jax
import jax.numpy as jnp
from jax import lax
from jax.experimental import pallas as pl
from jax.experimental.pallas import tpu as pltpu
from jax.experimental.pallas import tpu_sc as plsc

DIM = 32
NC = 2
NS = 16
NW = NC * NS

GATHER = 128
GPG = 10
GROUP = GATHER * GPG


def _emb_body(table_hbm, idx_hbm, out_hbm, idx_v, rows_v, sem):
    n_groups = idx_hbm.shape[1]
    wid = lax.axis_index("s") * NC + lax.axis_index("c")

    def group(g, carry):
        pltpu.sync_copy(idx_hbm.at[wid, g], idx_v)
        handles = [
            pltpu.async_copy(
                table_hbm.at[idx_v.at[j]],
                rows_v.at[pl.ds(j * GATHER, GATHER)],
                sem,
            )
            for j in range(GPG)
        ]
        for h in handles:
            h.wait()
        pltpu.sync_copy(rows_v, out_hbm.at[wid, g])
        return carry

    lax.fori_loop(0, n_groups, group, 0)


def _make_call(n_groups):
    return pl.kernel(
        _emb_body,
        out_type=jax.ShapeDtypeStruct((NW, n_groups, GROUP, DIM), jnp.float32),
        mesh=plsc.VectorSubcoreMesh(core_axis_name="c", subcore_axis_name="s"),
        scratch_types=[
            pltpu.VMEM((GPG, GATHER), jnp.int32),
            pltpu.VMEM((GROUP, DIM), jnp.float32),
            pltpu.SemaphoreType.DMA,
        ],
        compiler_params=pltpu.CompilerParams(use_tc_tiling_on_sc=False),
    )


def kernel(x, table):
    b, s = x.shape
    total = b * s
    assert total % (NW * GROUP) == 0
    n_groups = total // (NW * GROUP)
    idx = x.reshape(NW, n_groups, GPG, GATHER).astype(jnp.int32)
    out = _make_call(n_groups)(table, idx)
    return out.reshape(b, s, DIM)


# speedup vs baseline: 1.2836x; 1.0123x over previous
import jax
import jax.numpy as jnp
from jax import lax
from jax.experimental import pallas as pl
from jax.experimental.pallas import tpu as pltpu
from jax.experimental.pallas import tpu_sc as plsc

DIM = 32
NC = 2
NS = 16
NW = NC * NS

GATHER = 128
GPG = 10
GROUP = GATHER * GPG


def _emb_body(table_hbm, idx_hbm, out_hbm, idx_v, rows_v, semg0, semg1, semo0, semo1):
    n_groups = out_hbm.shape[1]
    wid = lax.axis_index("s") * NC + lax.axis_index("c")

    pltpu.sync_copy(idx_hbm.at[wid], idx_v)

    sems_g = (semg0, semg1)
    sems_o = (semo0, semo1)

    def fire(g, buf):
        for j in range(GPG):
            pltpu.async_copy(
                table_hbm.at[idx_v.at[g * GPG + j]],
                rows_v.at[buf, pl.ds(j * GATHER, GATHER)],
                sems_g[buf],
            )

    def drain_gathers(buf):
        pltpu.make_async_copy(
            table_hbm.at[pl.ds(0, GROUP)], rows_v.at[buf], sems_g[buf]
        ).wait()

    def wait_writeout(buf, g):
        pltpu.make_async_copy(
            rows_v.at[buf], out_hbm.at[wid, g], sems_o[buf]
        ).wait()

    def pair(p, carry):
        g0 = 2 * p
        g1 = 2 * p + 1

        @pl.when(p >= 1)
        def _():
            wait_writeout(0, g0 - 2)

        fire(g0, 0)

        @pl.when(p >= 1)
        def _():
            wait_writeout(1, g1 - 2)

        fire(g1, 1)

        drain_gathers(0)
        pltpu.async_copy(rows_v.at[0], out_hbm.at[wid, g0], sems_o[0])
        drain_gathers(1)
        pltpu.async_copy(rows_v.at[1], out_hbm.at[wid, g1], sems_o[1])
        return carry

    lax.fori_loop(0, n_groups // 2, pair, 0)
    wait_writeout(0, n_groups - 2)
    wait_writeout(1, n_groups - 1)


def _make_call(n_groups):
    return pl.kernel(
        _emb_body,
        out_type=jax.ShapeDtypeStruct((NW, n_groups, GROUP, DIM), jnp.float32),
        mesh=plsc.VectorSubcoreMesh(core_axis_name="c", subcore_axis_name="s"),
        scratch_types=[
            pltpu.VMEM((n_groups * GPG, GATHER), jnp.int32),
            pltpu.VMEM((2, GROUP, DIM), jnp.float32),
            pltpu.SemaphoreType.DMA,
            pltpu.SemaphoreType.DMA,
            pltpu.SemaphoreType.DMA,
            pltpu.SemaphoreType.DMA,
        ],
        compiler_params=pltpu.CompilerParams(use_tc_tiling_on_sc=False),
    )


def kernel(x, table):
    b, s = x.shape
    total = b * s
    assert total % (NW * GROUP) == 0
    n_groups = total // (NW * GROUP)
    assert n_groups % 2 == 0
    idx = x.reshape(NW, n_groups * GPG, GATHER).astype(jnp.int32)
    out = _make_call(n_groups)(table, idx)
    return out.reshape(b, s, DIM)
